# baseline (speedup 1.0000x reference)
import functools

import jax
import jax.numpy as jnp
from jax import lax
from jax.experimental import pallas as pl
from jax.experimental.pallas import tpu as pltpu
from jax.experimental.pallas import tpu_sc as plsc

N_NODES = 10000
N_PAD = 10240
G_POOL = 64
NC = 2
NS = 16
NW = NC * NS
CH = 128
E_EDGES = 320000
CPW0 = 60
CPW1 = 100
CHUNKS_PER_W = 80
CHUNKS_TOT = NS * (CPW0 + CPW1)
E_PAD = CHUNKS_TOT * CH
EW = CHUNKS_PER_W * CH
ROWS_PER_TILE = N_PAD // NS


def _make_count():
  mesh = plsc.VectorSubcoreMesh(core_axis_name="c", subcore_axis_name="s",
                                num_cores=NC, num_subcores=NS)
  out_type = jax.ShapeDtypeStruct((NW * N_PAD,), jnp.float32)
  scratch = [
      pltpu.VMEM((N_PAD,), jnp.float32),
      pltpu.VMEM((CH,), jnp.int32),
  ]

  def body(dst_hbm, cnt_out, cnt_vmem, dst_buf):
    cid = lax.axis_index("c")
    sid = lax.axis_index("s")
    wid = sid * NC + cid

    def zero(i, carry):
      cnt_vmem[pl.ds(i * 16, 16)] = jnp.zeros((16,), jnp.float32)
      return carry

    lax.fori_loop(0, N_PAD // 16, zero, 0)

    c0 = wid * CHUNKS_PER_W
    ones16 = jnp.ones((16,), jnp.float32)

    def chunk(j, carry):
      pltpu.sync_copy(dst_hbm.at[c0 + j], dst_buf)
      for k in range(CH // 16):
        idx = dst_buf[pl.ds(k * 16, 16)]
        plsc.addupdate_scatter(cnt_vmem, [idx], ones16)
      return carry

    lax.fori_loop(0, CHUNKS_PER_W, chunk, 0)
    pltpu.sync_copy(cnt_vmem, cnt_out.at[pl.ds(wid * N_PAD, N_PAD)])

  return pl.kernel(
      body, out_type=out_type, mesh=mesh, scratch_types=scratch,
      compiler_params=pltpu.CompilerParams(needs_layout_passes=False))


def _make_aggregate(D):
  mesh = plsc.VectorSubcoreMesh(core_axis_name="c", subcore_axis_name="s",
                                num_cores=NC, num_subcores=NS)
  out_type = jax.ShapeDtypeStruct((NC * N_PAD, D), jnp.float32)
  scratch = [
      pltpu.VMEM_SHARED((N_PAD, D), jnp.float32),
      pltpu.VMEM((CH,), jnp.int32),
      pltpu.VMEM((CH,), jnp.int32),
      pltpu.VMEM((CH, D), jnp.float32),
      pltpu.SemaphoreType.DMA,
  ]

  def body(x_hbm, src_hbm, dst_hbm, zf_hbm, part_out, acc, src_buf,
           dst_buf, rows, sem):
    cid = lax.axis_index("c")
    sid = lax.axis_index("s")
    r0 = sid * ROWS_PER_TILE
    c0 = lax.select(cid == 0, sid * CPW0, NS * CPW0 + sid * CPW1)
    ncw = lax.select(cid == 0, CPW0, CPW1)

    pltpu.sync_copy(zf_hbm.at[pl.ds(r0, ROWS_PER_TILE)],
                    acc.at[pl.ds(r0, ROWS_PER_TILE)])
    plsc.subcore_barrier()

    def chunk(j, carry):
      pltpu.sync_copy(src_hbm.at[c0 + j], src_buf)
      pltpu.sync_copy(dst_hbm.at[c0 + j], dst_buf)
      pltpu.async_copy(x_hbm.at[src_buf], rows, sem).wait()
      pltpu.sync_copy(rows, acc.at[dst_buf], add=True)
      return carry

    lax.fori_loop(0, ncw, chunk, 0)
    plsc.subcore_barrier()

    pltpu.sync_copy(acc.at[pl.ds(r0, ROWS_PER_TILE)],
                    part_out.at[pl.ds(cid * N_PAD + r0, ROWS_PER_TILE)])

  return pl.kernel(body, out_type=out_type, mesh=mesh, scratch_types=scratch)


def _dense_body(refs):
  part, cntp, x, wl, bl, wr, g, be, out = refs
  s = part[0] + part[1]
  cnt = jnp.sum(cntp[...], axis=0)[:, None]
  mean = s / jnp.maximum(cnt, 1.0)
  a = lax.dot_general(mean, wl[...], (((1,), (1,)), ((), ())),
                      preferred_element_type=jnp.float32)
  pre = a + bl[...][None, :] + lax.dot_general(
      x[...], wr[...], (((1,), (1,)), ((), ())),
      preferred_element_type=jnp.float32)

  mask = (lax.broadcasted_iota(jnp.int32, (N_PAD, 1), 0)
          < N_NODES).astype(jnp.float32)
  m = jnp.sum(pre * mask, axis=0, keepdims=True) / N_NODES
  d = pre - m
  var = jnp.sum((pre - m) * (pre - m) * mask, axis=0, keepdims=True) / N_NODES
  h = jnp.maximum(d / jnp.sqrt(var + 1e-5) * g[...][None, :]
                  + be[...][None, :], 0.0) * mask
  out[...] = h


def _make_dense(dout):
  outs = jax.ShapeDtypeStruct((N_PAD, dout), jnp.float32)

  def kern(*refs):
    _dense_body(refs)

  return pl.pallas_call(
      kern, out_shape=outs,
      compiler_params=pltpu.CompilerParams(vmem_limit_bytes=100 * 1024 * 1024))


def _pool_body(h3, batch, fc1w, fc1b, fc2w, fc2b, out):
  onehot = (batch[...][:, None]
            == lax.broadcasted_iota(jnp.int32, (1, G_POOL), 1)
            ).astype(jnp.float32)
  s = lax.dot_general(onehot, h3[...], (((0,), (0,)), ((), ())),
                      preferred_element_type=jnp.float32,
                      precision=lax.Precision.HIGHEST)
  c = jnp.sum(onehot, axis=0)[:, None]
  pooled = s / jnp.maximum(c, 1.0)
  z = jnp.maximum(
      lax.dot_general(pooled, fc1w[...], (((1,), (1,)), ((), ())),
                      preferred_element_type=jnp.float32)
      + fc1b[...][None, :], 0.0)
  out[...] = (lax.dot_general(z, fc2w[...], (((1,), (1,)), ((), ())),
                              preferred_element_type=jnp.float32)
              + fc2b[...][None, :])


_count = _make_count()
_agg128 = _make_aggregate(128)
_dense128 = _make_dense(128)
_dense64 = _make_dense(64)
_pool = pl.pallas_call(
    _pool_body, out_shape=jax.ShapeDtypeStruct((G_POOL, 2), jnp.float32))


@jax.jit
def kernel(x, edge_index, batch, Wl1, bl1, Wr1, g1, be1, Wl2, bl2, Wr2, g2,
           be2, Wl3, bl3, Wr3, g3, be3, fc1_w, fc1_b, fc2_w, fc2_b):
  src = jnp.concatenate(
      [edge_index[0], jnp.zeros((E_PAD - E_EDGES,), jnp.int32)])
  src = src.reshape(NW * CHUNKS_PER_W, CH)
  dst = jnp.concatenate(
      [edge_index[1], jnp.full((E_PAD - E_EDGES,), N_PAD - 1, jnp.int32)])
  dst = dst.reshape(NW * CHUNKS_PER_W, CH)
  pad_n = N_PAD - x.shape[0]
  x_p = jnp.pad(x, ((0, pad_n), (0, 0)))
  batch_p = jnp.pad(batch, (0, pad_n), constant_values=G_POOL)
  zeros128 = jnp.zeros((N_PAD, 128), jnp.float32)

  cnt = _count(dst).reshape(NW, N_PAD)
  part1 = _agg128(x_p, src, dst, zeros128).reshape(NC, N_PAD, 128)
  h1 = _dense128(part1, cnt, x_p, Wl1, bl1, Wr1, g1, be1)
  part2 = _agg128(h1, src, dst, zeros128).reshape(NC, N_PAD, 128)
  h2 = _dense128(part2, cnt, h1, Wl2, bl2, Wr2, g2, be2)
  part3 = _agg128(h2, src, dst, zeros128).reshape(NC, N_PAD, 128)
  h3 = _dense64(part3, cnt, h2, Wl3, bl3, Wr3, g3, be3)
  return _pool(h3, batch_p, fc1_w, fc1_b, fc2_w, fc2_b)


# speedup vs baseline: 1.0715x; 1.0715x over previous
import functools

import jax
import jax.numpy as jnp
from jax import lax
from jax.experimental import pallas as pl
from jax.experimental.pallas import tpu as pltpu
from jax.experimental.pallas import tpu_sc as plsc

N_NODES = 10000
N_PAD = 10240
G_POOL = 64
NC = 2
NS = 16
NW = NC * NS
CH = 128
E_EDGES = 320000
CPW0 = 60
CPW1 = 100
CHUNKS_PER_W = 80
CHUNKS_TOT = NS * (CPW0 + CPW1)
E_PAD = CHUNKS_TOT * CH
EW = CHUNKS_PER_W * CH
ROWS_PER_TILE = N_PAD // NS


def _make_count():
  mesh = plsc.VectorSubcoreMesh(core_axis_name="c", subcore_axis_name="s",
                                num_cores=NC, num_subcores=NS)
  out_type = jax.ShapeDtypeStruct((NW * N_PAD,), jnp.float32)
  scratch = [
      pltpu.VMEM((N_PAD,), jnp.float32),
      pltpu.VMEM((CH,), jnp.int32),
  ]

  def body(dst_hbm, cnt_out, cnt_vmem, dst_buf):
    cid = lax.axis_index("c")
    sid = lax.axis_index("s")
    wid = sid * NC + cid

    def zero(i, carry):
      cnt_vmem[pl.ds(i * 16, 16)] = jnp.zeros((16,), jnp.float32)
      return carry

    lax.fori_loop(0, N_PAD // 16, zero, 0)

    c0 = wid * CHUNKS_PER_W
    ones16 = jnp.ones((16,), jnp.float32)

    def chunk(j, carry):
      pltpu.sync_copy(dst_hbm.at[c0 + j], dst_buf)
      for k in range(CH // 16):
        idx = dst_buf[pl.ds(k * 16, 16)]
        plsc.addupdate_scatter(cnt_vmem, [idx], ones16)
      return carry

    lax.fori_loop(0, CHUNKS_PER_W, chunk, 0)
    pltpu.sync_copy(cnt_vmem, cnt_out.at[pl.ds(wid * N_PAD, N_PAD)])

  return pl.kernel(
      body, out_type=out_type, mesh=mesh, scratch_types=scratch,
      compiler_params=pltpu.CompilerParams(needs_layout_passes=False))


def _make_aggregate(D):
  mesh = plsc.VectorSubcoreMesh(core_axis_name="c", subcore_axis_name="s",
                                num_cores=NC, num_subcores=NS)
  out_type = jax.ShapeDtypeStruct((NC * N_PAD, D), jnp.float32)
  scratch = [
      pltpu.VMEM_SHARED((N_PAD, D), jnp.float32),
      pltpu.VMEM((CH,), jnp.int32),
      pltpu.VMEM((CH,), jnp.int32),
      pltpu.VMEM((CH, D), jnp.float32),
      pltpu.SemaphoreType.DMA,
  ]

  ZCH = ROWS_PER_TILE // CH

  def body(x_hbm, src_hbm, dst_hbm, zf_hbm, part_out, acc, src_buf,
           dst_buf, rows, sem):
    cid = lax.axis_index("c")
    sid = lax.axis_index("s")
    wid = sid * NC + cid
    r0 = sid * ROWS_PER_TILE
    c0 = wid * CHUNKS_PER_W

    def zchunk(k, carry):
      rr = r0 + k * CH
      pltpu.sync_copy(zf_hbm.at[pl.ds(rr, CH)], rows)
      pltpu.sync_copy(rows, acc.at[pl.ds(rr, CH)])
      return carry

    lax.fori_loop(0, ZCH, zchunk, 0)
    plsc.subcore_barrier()

    def chunk(j, carry):
      pltpu.sync_copy(src_hbm.at[c0 + j], src_buf)
      pltpu.sync_copy(dst_hbm.at[c0 + j], dst_buf)
      pltpu.async_copy(x_hbm.at[src_buf], rows, sem).wait()
      pltpu.sync_copy(rows, acc.at[dst_buf], add=True)
      return carry

    lax.fori_loop(0, CHUNKS_PER_W, chunk, 0)
    plsc.subcore_barrier()

    def ochunk(k, carry):
      rr = r0 + k * CH
      pltpu.sync_copy(acc.at[pl.ds(rr, CH)], rows)
      pltpu.sync_copy(rows, part_out.at[pl.ds(cid * N_PAD + rr, CH)])
      return carry

    lax.fori_loop(0, ZCH, ochunk, 0)

  return pl.kernel(body, out_type=out_type, mesh=mesh, scratch_types=scratch)


def _dense_body(refs):
  part, cntp, x, wl, bl, wr, g, be, out = refs
  s = part[0] + part[1]
  cnt = jnp.sum(cntp[...], axis=0)[:, None]
  mean = s / jnp.maximum(cnt, 1.0)
  a = lax.dot_general(mean, wl[...], (((1,), (1,)), ((), ())),
                      preferred_element_type=jnp.float32)
  pre = a + bl[...][None, :] + lax.dot_general(
      x[...], wr[...], (((1,), (1,)), ((), ())),
      preferred_element_type=jnp.float32)

  mask = (lax.broadcasted_iota(jnp.int32, (N_PAD, 1), 0)
          < N_NODES).astype(jnp.float32)
  m = jnp.sum(pre * mask, axis=0, keepdims=True) / N_NODES
  d = pre - m
  var = jnp.sum((pre - m) * (pre - m) * mask, axis=0, keepdims=True) / N_NODES
  h = jnp.maximum(d / jnp.sqrt(var + 1e-5) * g[...][None, :]
                  + be[...][None, :], 0.0) * mask
  out[...] = h


def _make_dense(dout):
  outs = jax.ShapeDtypeStruct((N_PAD, dout), jnp.float32)

  def kern(*refs):
    _dense_body(refs)

  return pl.pallas_call(
      kern, out_shape=outs,
      compiler_params=pltpu.CompilerParams(vmem_limit_bytes=100 * 1024 * 1024))


def _pool_body(h3, batch, fc1w, fc1b, fc2w, fc2b, out):
  onehot = (batch[...][:, None]
            == lax.broadcasted_iota(jnp.int32, (1, G_POOL), 1)
            ).astype(jnp.float32)
  s = lax.dot_general(onehot, h3[...], (((0,), (0,)), ((), ())),
                      preferred_element_type=jnp.float32,
                      precision=lax.Precision.HIGHEST)
  c = jnp.sum(onehot, axis=0)[:, None]
  pooled = s / jnp.maximum(c, 1.0)
  z = jnp.maximum(
      lax.dot_general(pooled, fc1w[...], (((1,), (1,)), ((), ())),
                      preferred_element_type=jnp.float32)
      + fc1b[...][None, :], 0.0)
  out[...] = (lax.dot_general(z, fc2w[...], (((1,), (1,)), ((), ())),
                              preferred_element_type=jnp.float32)
              + fc2b[...][None, :])


_count = _make_count()
_agg128 = _make_aggregate(128)
_dense128 = _make_dense(128)
_dense64 = _make_dense(64)
_pool = pl.pallas_call(
    _pool_body, out_shape=jax.ShapeDtypeStruct((G_POOL, 2), jnp.float32))


@jax.jit
def kernel(x, edge_index, batch, Wl1, bl1, Wr1, g1, be1, Wl2, bl2, Wr2, g2,
           be2, Wl3, bl3, Wr3, g3, be3, fc1_w, fc1_b, fc2_w, fc2_b):
  src = jnp.concatenate(
      [edge_index[0], jnp.zeros((E_PAD - E_EDGES,), jnp.int32)])
  src = src.reshape(NW * CHUNKS_PER_W, CH)
  dst = jnp.concatenate(
      [edge_index[1], jnp.full((E_PAD - E_EDGES,), N_PAD - 1, jnp.int32)])
  dst = dst.reshape(NW * CHUNKS_PER_W, CH)
  pad_n = N_PAD - x.shape[0]
  x_p = jnp.pad(x, ((0, pad_n), (0, 0)))
  batch_p = jnp.pad(batch, (0, pad_n), constant_values=G_POOL)
  zeros128 = jnp.zeros((N_PAD, 128), jnp.float32)

  cnt = _count(dst).reshape(NW, N_PAD)
  part1 = _agg128(x_p, src, dst, zeros128).reshape(NC, N_PAD, 128)
  h1 = _dense128(part1, cnt, x_p, Wl1, bl1, Wr1, g1, be1)
  part2 = _agg128(h1, src, dst, zeros128).reshape(NC, N_PAD, 128)
  h2 = _dense128(part2, cnt, h1, Wl2, bl2, Wr2, g2, be2)
  part3 = _agg128(h2, src, dst, zeros128).reshape(NC, N_PAD, 128)
  h3 = _dense64(part3, cnt, h2, Wl3, bl3, Wr3, g3, be3)
  return _pool(h3, batch_p, fc1_w, fc1_b, fc2_w, fc2_b)


# speedup vs baseline: 1.1719x; 1.0937x over previous
import functools

import jax
import jax.numpy as jnp
from jax import lax
from jax.experimental import pallas as pl
from jax.experimental.pallas import tpu as pltpu
from jax.experimental.pallas import tpu_sc as plsc

N_NODES = 10000
N_PAD = 10240
G_POOL = 64
NC = 2
NS = 16
NW = NC * NS
CH = 128
E_EDGES = 320000
CHUNKS_PER_W = 80
CHUNKS_TOT = NW * CHUNKS_PER_W
E_PAD = CHUNKS_TOT * CH
EW = CHUNKS_PER_W * CH
ROWS_PER_TILE = N_PAD // NS


def _make_count():
  mesh = plsc.VectorSubcoreMesh(core_axis_name="c", subcore_axis_name="s",
                                num_cores=NC, num_subcores=NS)
  out_type = jax.ShapeDtypeStruct((NW * N_PAD,), jnp.float32)
  scratch = [
      pltpu.VMEM((N_PAD,), jnp.float32),
      pltpu.VMEM((CH,), jnp.int32),
  ]

  def body(dst_hbm, cnt_out, cnt_vmem, dst_buf):
    cid = lax.axis_index("c")
    sid = lax.axis_index("s")
    wid = sid * NC + cid

    def zero(i, carry):
      cnt_vmem[pl.ds(i * 16, 16)] = jnp.zeros((16,), jnp.float32)
      return carry

    lax.fori_loop(0, N_PAD // 16, zero, 0)

    base = wid * EW
    ones16 = jnp.ones((16,), jnp.float32)

    def chunk(j, carry):
      pltpu.sync_copy(dst_hbm.at[pl.ds(base + j * CH, CH)], dst_buf)
      for k in range(CH // 16):
        idx = dst_buf[pl.ds(k * 16, 16)]
        plsc.addupdate_scatter(cnt_vmem, [idx], ones16)
      return carry

    lax.fori_loop(0, CHUNKS_PER_W, chunk, 0)
    pltpu.sync_copy(cnt_vmem, cnt_out.at[pl.ds(wid * N_PAD, N_PAD)])

  return pl.kernel(
      body, out_type=out_type, mesh=mesh, scratch_types=scratch,
      compiler_params=pltpu.CompilerParams(needs_layout_passes=False))


def _make_aggregate(D):
  mesh = plsc.VectorSubcoreMesh(core_axis_name="c", subcore_axis_name="s",
                                num_cores=NC, num_subcores=NS)
  out_type = jax.ShapeDtypeStruct((NC * N_PAD, D), jnp.float32)
  scratch = [
      pltpu.VMEM_SHARED((N_PAD, D), jnp.float32),
      pltpu.VMEM((CH,), jnp.int32),
      pltpu.VMEM((CH,), jnp.int32),
      pltpu.VMEM((CH, D), jnp.float32),
      pltpu.SemaphoreType.DMA,
  ]

  ZCH = ROWS_PER_TILE // CH

  def body(x_hbm, src_hbm, dst_hbm, zf_hbm, part_out, acc, src_buf,
           dst_buf, rows, sem):
    cid = lax.axis_index("c")
    sid = lax.axis_index("s")
    wid = sid * NC + cid
    r0 = sid * ROWS_PER_TILE
    base = wid * EW

    def zchunk(k, carry):
      rr = r0 + k * CH
      pltpu.sync_copy(zf_hbm.at[pl.ds(rr, CH)], rows)
      pltpu.sync_copy(rows, acc.at[pl.ds(rr, CH)])
      return carry

    lax.fori_loop(0, ZCH, zchunk, 0)
    plsc.subcore_barrier()

    def chunk(j, carry):
      off = base + j * CH
      pltpu.sync_copy(src_hbm.at[pl.ds(off, CH)], src_buf)
      pltpu.sync_copy(dst_hbm.at[pl.ds(off, CH)], dst_buf)
      pltpu.async_copy(x_hbm.at[src_buf], rows, sem).wait()
      pltpu.sync_copy(rows, acc.at[dst_buf], add=True)
      return carry

    lax.fori_loop(0, CHUNKS_PER_W, chunk, 0)
    plsc.subcore_barrier()

    def ochunk(k, carry):
      rr = r0 + k * CH
      pltpu.sync_copy(acc.at[pl.ds(rr, CH)], rows)
      pltpu.sync_copy(rows, part_out.at[pl.ds(cid * N_PAD + rr, CH)])
      return carry

    lax.fori_loop(0, ZCH, ochunk, 0)

  return pl.kernel(body, out_type=out_type, mesh=mesh, scratch_types=scratch)


def _dense_body(refs):
  part, cntp, x, wl, bl, wr, g, be, out = refs
  s = part[0] + part[1]
  cnt = jnp.sum(cntp[...], axis=0)[:, None]
  mean = s / jnp.maximum(cnt, 1.0)
  a = lax.dot_general(mean, wl[...], (((1,), (1,)), ((), ())),
                      preferred_element_type=jnp.float32)
  pre = a + bl[...][None, :] + lax.dot_general(
      x[...], wr[...], (((1,), (1,)), ((), ())),
      preferred_element_type=jnp.float32)

  mask = (lax.broadcasted_iota(jnp.int32, (N_PAD, 1), 0)
          < N_NODES).astype(jnp.float32)
  m = jnp.sum(pre * mask, axis=0, keepdims=True) / N_NODES
  d = pre - m
  var = jnp.sum((pre - m) * (pre - m) * mask, axis=0, keepdims=True) / N_NODES
  h = jnp.maximum(d / jnp.sqrt(var + 1e-5) * g[...][None, :]
                  + be[...][None, :], 0.0) * mask
  out[...] = h


def _make_dense(dout):
  outs = jax.ShapeDtypeStruct((N_PAD, dout), jnp.float32)

  def kern(*refs):
    _dense_body(refs)

  return pl.pallas_call(
      kern, out_shape=outs,
      compiler_params=pltpu.CompilerParams(vmem_limit_bytes=100 * 1024 * 1024))


def _pool_body(h3, batch, fc1w, fc1b, fc2w, fc2b, out):
  onehot = (batch[...][:, None]
            == lax.broadcasted_iota(jnp.int32, (1, G_POOL), 1)
            ).astype(jnp.float32)
  s = lax.dot_general(onehot, h3[...], (((0,), (0,)), ((), ())),
                      preferred_element_type=jnp.float32,
                      precision=lax.Precision.HIGHEST)
  c = jnp.sum(onehot, axis=0)[:, None]
  pooled = s / jnp.maximum(c, 1.0)
  z = jnp.maximum(
      lax.dot_general(pooled, fc1w[...], (((1,), (1,)), ((), ())),
                      preferred_element_type=jnp.float32)
      + fc1b[...][None, :], 0.0)
  out[...] = (lax.dot_general(z, fc2w[...], (((1,), (1,)), ((), ())),
                              preferred_element_type=jnp.float32)
              + fc2b[...][None, :])


_count = _make_count()
_agg128 = _make_aggregate(128)
_dense128 = _make_dense(128)
_dense64 = _make_dense(64)
_pool = pl.pallas_call(
    _pool_body, out_shape=jax.ShapeDtypeStruct((G_POOL, 2), jnp.float32))


@jax.jit
def kernel(x, edge_index, batch, Wl1, bl1, Wr1, g1, be1, Wl2, bl2, Wr2, g2,
           be2, Wl3, bl3, Wr3, g3, be3, fc1_w, fc1_b, fc2_w, fc2_b):
  src = jnp.concatenate(
      [edge_index[0], jnp.zeros((E_PAD - E_EDGES,), jnp.int32)])
  dst = jnp.concatenate(
      [edge_index[1], jnp.full((E_PAD - E_EDGES,), N_PAD - 1, jnp.int32)])
  pad_n = N_PAD - x.shape[0]
  x_p = jnp.pad(x, ((0, pad_n), (0, 0)))
  batch_p = jnp.pad(batch, (0, pad_n), constant_values=G_POOL)
  zeros128 = jnp.zeros((N_PAD, 128), jnp.float32)

  cnt = _count(dst).reshape(NW, N_PAD)
  part1 = _agg128(x_p, src, dst, zeros128).reshape(NC, N_PAD, 128)
  h1 = _dense128(part1, cnt, x_p, Wl1, bl1, Wr1, g1, be1)
  part2 = _agg128(h1, src, dst, zeros128).reshape(NC, N_PAD, 128)
  h2 = _dense128(part2, cnt, h1, Wl2, bl2, Wr2, g2, be2)
  part3 = _agg128(h2, src, dst, zeros128).reshape(NC, N_PAD, 128)
  h3 = _dense64(part3, cnt, h2, Wl3, bl3, Wr3, g3, be3)
  return _pool(h3, batch_p, fc1_w, fc1_b, fc2_w, fc2_b)


# speedup vs baseline: 1.3094x; 1.1173x over previous
import functools

import jax
import jax.numpy as jnp
from jax import lax
from jax.experimental import pallas as pl
from jax.experimental.pallas import tpu as pltpu
from jax.experimental.pallas import tpu_sc as plsc

N_NODES = 10000
N_PAD = 10240
G_POOL = 64
NC = 2
NS = 16
NW = NC * NS
CH = 128
E_EDGES = 320000
CHUNKS_PER_W = 80
CHUNKS_TOT = NW * CHUNKS_PER_W
E_PAD = CHUNKS_TOT * CH
EW = CHUNKS_PER_W * CH
ROWS_PER_TILE = N_PAD // NS


def _make_count():
  mesh = plsc.VectorSubcoreMesh(core_axis_name="c", subcore_axis_name="s",
                                num_cores=NC, num_subcores=NS)
  out_type = jax.ShapeDtypeStruct((NW * N_PAD,), jnp.float32)
  scratch = [
      pltpu.VMEM((N_PAD,), jnp.float32),
      pltpu.VMEM((CH,), jnp.int32),
  ]

  def body(dst_hbm, cnt_out, cnt_vmem, dst_buf):
    cid = lax.axis_index("c")
    sid = lax.axis_index("s")
    wid = sid * NC + cid

    def zero(i, carry):
      cnt_vmem[pl.ds(i * 16, 16)] = jnp.zeros((16,), jnp.float32)
      return carry

    lax.fori_loop(0, N_PAD // 16, zero, 0)

    base = wid * EW
    ones16 = jnp.ones((16,), jnp.float32)

    def chunk(j, carry):
      pltpu.sync_copy(dst_hbm.at[pl.ds((j * NW + wid) * CH, CH)], dst_buf)
      for k in range(CH // 16):
        idx = dst_buf[pl.ds(k * 16, 16)]
        plsc.addupdate_scatter(cnt_vmem, [idx], ones16)
      return carry

    lax.fori_loop(0, CHUNKS_PER_W, chunk, 0)
    pltpu.sync_copy(cnt_vmem, cnt_out.at[pl.ds(wid * N_PAD, N_PAD)])

  return pl.kernel(
      body, out_type=out_type, mesh=mesh, scratch_types=scratch,
      compiler_params=pltpu.CompilerParams(needs_layout_passes=False))


def _make_aggregate(D):
  mesh = plsc.VectorSubcoreMesh(core_axis_name="c", subcore_axis_name="s",
                                num_cores=NC, num_subcores=NS)
  out_type = jax.ShapeDtypeStruct((NC * N_PAD, D), jnp.float32)
  scratch = [
      pltpu.VMEM_SHARED((N_PAD, D), jnp.float32),
      pltpu.VMEM((CH,), jnp.int32),
      pltpu.VMEM((CH,), jnp.int32),
      pltpu.VMEM((CH, D), jnp.float32),
      pltpu.SemaphoreType.DMA,
  ]

  ZCH = ROWS_PER_TILE // CH

  def body(x_hbm, src_hbm, dst_hbm, zf_hbm, part_out, acc, src_buf,
           dst_buf, rows, sem):
    cid = lax.axis_index("c")
    sid = lax.axis_index("s")
    wid = sid * NC + cid
    r0 = sid * ROWS_PER_TILE
    base = wid * EW

    def zchunk(k, carry):
      rr = r0 + k * CH
      pltpu.sync_copy(zf_hbm.at[pl.ds(rr, CH)], rows)
      pltpu.sync_copy(rows, acc.at[pl.ds(rr, CH)])
      return carry

    lax.fori_loop(0, ZCH, zchunk, 0)
    plsc.subcore_barrier()

    def chunk(j, carry):
      off = (j * NW + wid) * CH
      pltpu.sync_copy(src_hbm.at[pl.ds(off, CH)], src_buf)
      pltpu.sync_copy(dst_hbm.at[pl.ds(off, CH)], dst_buf)
      pltpu.async_copy(x_hbm.at[src_buf], rows, sem).wait()
      pltpu.sync_copy(rows, acc.at[dst_buf], add=True)
      return carry

    lax.fori_loop(0, CHUNKS_PER_W, chunk, 0)
    plsc.subcore_barrier()

    def ochunk(k, carry):
      rr = r0 + k * CH
      pltpu.sync_copy(acc.at[pl.ds(rr, CH)], rows)
      pltpu.sync_copy(rows, part_out.at[pl.ds(cid * N_PAD + rr, CH)])
      return carry

    lax.fori_loop(0, ZCH, ochunk, 0)

  return pl.kernel(body, out_type=out_type, mesh=mesh, scratch_types=scratch)


def _dense_body(refs):
  part, cntp, x, wl, bl, wr, g, be, out = refs
  s = part[0] + part[1]
  cnt = jnp.sum(cntp[...], axis=0)[:, None]
  mean = s / jnp.maximum(cnt, 1.0)
  a = lax.dot_general(mean, wl[...], (((1,), (1,)), ((), ())),
                      preferred_element_type=jnp.float32)
  pre = a + bl[...][None, :] + lax.dot_general(
      x[...], wr[...], (((1,), (1,)), ((), ())),
      preferred_element_type=jnp.float32)

  mask = (lax.broadcasted_iota(jnp.int32, (N_PAD, 1), 0)
          < N_NODES).astype(jnp.float32)
  m = jnp.sum(pre * mask, axis=0, keepdims=True) / N_NODES
  d = pre - m
  var = jnp.sum((pre - m) * (pre - m) * mask, axis=0, keepdims=True) / N_NODES
  h = jnp.maximum(d / jnp.sqrt(var + 1e-5) * g[...][None, :]
                  + be[...][None, :], 0.0) * mask
  out[...] = h


def _make_dense(dout):
  outs = jax.ShapeDtypeStruct((N_PAD, dout), jnp.float32)

  def kern(*refs):
    _dense_body(refs)

  return pl.pallas_call(
      kern, out_shape=outs,
      compiler_params=pltpu.CompilerParams(vmem_limit_bytes=100 * 1024 * 1024))


def _pool_body(h3, batch, fc1w, fc1b, fc2w, fc2b, out):
  onehot = (batch[...][:, None]
            == lax.broadcasted_iota(jnp.int32, (1, G_POOL), 1)
            ).astype(jnp.float32)
  s = lax.dot_general(onehot, h3[...], (((0,), (0,)), ((), ())),
                      preferred_element_type=jnp.float32,
                      precision=lax.Precision.HIGHEST)
  c = jnp.sum(onehot, axis=0)[:, None]
  pooled = s / jnp.maximum(c, 1.0)
  z = jnp.maximum(
      lax.dot_general(pooled, fc1w[...], (((1,), (1,)), ((), ())),
                      preferred_element_type=jnp.float32)
      + fc1b[...][None, :], 0.0)
  out[...] = (lax.dot_general(z, fc2w[...], (((1,), (1,)), ((), ())),
                              preferred_element_type=jnp.float32)
              + fc2b[...][None, :])


_count = _make_count()
_agg128 = _make_aggregate(128)
_dense128 = _make_dense(128)
_dense64 = _make_dense(64)
_pool = pl.pallas_call(
    _pool_body, out_shape=jax.ShapeDtypeStruct((G_POOL, 2), jnp.float32))


@jax.jit
def kernel(x, edge_index, batch, Wl1, bl1, Wr1, g1, be1, Wl2, bl2, Wr2, g2,
           be2, Wl3, bl3, Wr3, g3, be3, fc1_w, fc1_b, fc2_w, fc2_b):
  src = jnp.concatenate(
      [edge_index[0], jnp.zeros((E_PAD - E_EDGES,), jnp.int32)])
  dst = jnp.concatenate(
      [edge_index[1], jnp.full((E_PAD - E_EDGES,), N_PAD - 1, jnp.int32)])
  pad_n = N_PAD - x.shape[0]
  x_p = jnp.pad(x, ((0, pad_n), (0, 0)))
  batch_p = jnp.pad(batch, (0, pad_n), constant_values=G_POOL)
  zeros128 = jnp.zeros((N_PAD, 128), jnp.float32)

  cnt = _count(dst).reshape(NW, N_PAD)
  part1 = _agg128(x_p, src, dst, zeros128).reshape(NC, N_PAD, 128)
  h1 = _dense128(part1, cnt, x_p, Wl1, bl1, Wr1, g1, be1)
  part2 = _agg128(h1, src, dst, zeros128).reshape(NC, N_PAD, 128)
  h2 = _dense128(part2, cnt, h1, Wl2, bl2, Wr2, g2, be2)
  part3 = _agg128(h2, src, dst, zeros128).reshape(NC, N_PAD, 128)
  h3 = _dense64(part3, cnt, h2, Wl3, bl3, Wr3, g3, be3)
  return _pool(h3, batch_p, fc1_w, fc1_b, fc2_w, fc2_b)


# speedup vs baseline: 1.6344x; 1.2482x over previous
import functools

import jax
import jax.numpy as jnp
from jax import lax
from jax.experimental import pallas as pl
from jax.experimental.pallas import tpu as pltpu
from jax.experimental.pallas import tpu_sc as plsc

N_NODES = 10000
N_PAD = 10240
G_POOL = 64
NC = 2
NS = 16
NW = NC * NS
CH = 128
E_EDGES = 320000
CHUNKS_PER_W = 79
CHUNKS_TOT = NW * CHUNKS_PER_W
E_PAD = CHUNKS_TOT * CH
EW = CHUNKS_PER_W * CH
ROWS_PER_TILE = N_PAD // NS


def _make_count():
  mesh = plsc.VectorSubcoreMesh(core_axis_name="c", subcore_axis_name="s",
                                num_cores=NC, num_subcores=NS)
  out_type = jax.ShapeDtypeStruct((NW * N_PAD,), jnp.float32)
  scratch = [
      pltpu.VMEM((N_PAD,), jnp.float32),
      pltpu.VMEM((CH,), jnp.int32),
  ]

  def body(dst_hbm, cnt_out, cnt_vmem, dst_buf):
    cid = lax.axis_index("c")
    sid = lax.axis_index("s")
    wid = sid * NC + cid

    def zero(i, carry):
      cnt_vmem[pl.ds(i * 16, 16)] = jnp.zeros((16,), jnp.float32)
      return carry

    lax.fori_loop(0, N_PAD // 16, zero, 0)

    base = wid * EW
    ones16 = jnp.ones((16,), jnp.float32)

    def chunk(j, carry):
      pltpu.sync_copy(dst_hbm.at[pl.ds(base + j * CH, CH)], dst_buf)
      for k in range(CH // 16):
        idx = dst_buf[pl.ds(k * 16, 16)]
        plsc.addupdate_scatter(cnt_vmem, [idx], ones16)
      return carry

    lax.fori_loop(0, CHUNKS_PER_W, chunk, 0)
    pltpu.sync_copy(cnt_vmem, cnt_out.at[pl.ds(wid * N_PAD, N_PAD)])

  return pl.kernel(
      body, out_type=out_type, mesh=mesh, scratch_types=scratch,
      compiler_params=pltpu.CompilerParams(needs_layout_passes=False))


def _make_aggregate(D):
  mesh = plsc.VectorSubcoreMesh(core_axis_name="c", subcore_axis_name="s",
                                num_cores=NC, num_subcores=NS)
  out_type = jax.ShapeDtypeStruct((NC * N_PAD, D), jnp.float32)
  scratch = [
      pltpu.VMEM_SHARED((N_PAD, D), jnp.float32),
      pltpu.VMEM((CH,), jnp.int32),
      pltpu.VMEM((CH,), jnp.int32),
      pltpu.VMEM((CH, D), jnp.float32),
      pltpu.SemaphoreType.DMA,
  ]

  ZCH = ROWS_PER_TILE // CH

  def body(x_hbm, src_hbm, dst_hbm, zf_hbm, part_out, acc, src_buf,
           dst_buf, rows, sem):
    cid = lax.axis_index("c")
    sid = lax.axis_index("s")
    wid = sid * NC + cid
    r0 = sid * ROWS_PER_TILE
    base = wid * EW

    def zchunk(k, carry):
      rr = r0 + k * CH
      pltpu.sync_copy(zf_hbm.at[pl.ds(rr, CH)], rows)
      pltpu.sync_copy(rows, acc.at[pl.ds(rr, CH)])
      return carry

    lax.fori_loop(0, ZCH, zchunk, 0)
    plsc.subcore_barrier()

    def chunk(j, carry):
      off = base + j * CH
      pltpu.sync_copy(src_hbm.at[pl.ds(off, CH)], src_buf)
      pltpu.sync_copy(dst_hbm.at[pl.ds(off, CH)], dst_buf)
      pltpu.async_copy(x_hbm.at[src_buf], rows, sem).wait()
      pltpu.sync_copy(rows, acc.at[dst_buf], add=True)
      return carry

    lax.fori_loop(0, CHUNKS_PER_W, chunk, 0)
    plsc.subcore_barrier()

    def ochunk(k, carry):
      rr = r0 + k * CH
      pltpu.sync_copy(acc.at[pl.ds(rr, CH)], rows)
      pltpu.sync_copy(rows, part_out.at[pl.ds(cid * N_PAD + rr, CH)])
      return carry

    lax.fori_loop(0, ZCH, ochunk, 0)

  return pl.kernel(body, out_type=out_type, mesh=mesh, scratch_types=scratch)


def _dense_body(refs):
  part, cntp, x, wl, bl, wr, g, be, out = refs
  s = part[0] + part[1]
  cnt = jnp.sum(cntp[...], axis=0)[:, None]
  mean = s / jnp.maximum(cnt, 1.0)
  a = lax.dot_general(mean, wl[...], (((1,), (1,)), ((), ())),
                      preferred_element_type=jnp.float32)
  pre = a + bl[...][None, :] + lax.dot_general(
      x[...], wr[...], (((1,), (1,)), ((), ())),
      preferred_element_type=jnp.float32)

  mask = (lax.broadcasted_iota(jnp.int32, (N_PAD, 1), 0)
          < N_NODES).astype(jnp.float32)
  m = jnp.sum(pre * mask, axis=0, keepdims=True) / N_NODES
  d = pre - m
  var = jnp.sum((pre - m) * (pre - m) * mask, axis=0, keepdims=True) / N_NODES
  h = jnp.maximum(d / jnp.sqrt(var + 1e-5) * g[...][None, :]
                  + be[...][None, :], 0.0) * mask
  out[...] = h


def _make_dense(dout):
  outs = jax.ShapeDtypeStruct((N_PAD, dout), jnp.float32)

  def kern(*refs):
    _dense_body(refs)

  return pl.pallas_call(
      kern, out_shape=outs,
      compiler_params=pltpu.CompilerParams(vmem_limit_bytes=100 * 1024 * 1024))


def _pool_body(h3, batch, fc1w, fc1b, fc2w, fc2b, out):
  onehot = (batch[...][:, None]
            == lax.broadcasted_iota(jnp.int32, (1, G_POOL), 1)
            ).astype(jnp.float32)
  s = lax.dot_general(onehot, h3[...], (((0,), (0,)), ((), ())),
                      preferred_element_type=jnp.float32,
                      precision=lax.Precision.HIGHEST)
  c = jnp.sum(onehot, axis=0)[:, None]
  pooled = s / jnp.maximum(c, 1.0)
  z = jnp.maximum(
      lax.dot_general(pooled, fc1w[...], (((1,), (1,)), ((), ())),
                      preferred_element_type=jnp.float32)
      + fc1b[...][None, :], 0.0)
  out[...] = (lax.dot_general(z, fc2w[...], (((1,), (1,)), ((), ())),
                              preferred_element_type=jnp.float32)
              + fc2b[...][None, :])


_count = _make_count()
_agg128 = _make_aggregate(128)
_dense128 = _make_dense(128)
_dense64 = _make_dense(64)
_pool = pl.pallas_call(
    _pool_body, out_shape=jax.ShapeDtypeStruct((G_POOL, 2), jnp.float32))


@jax.jit
def kernel(x, edge_index, batch, Wl1, bl1, Wr1, g1, be1, Wl2, bl2, Wr2, g2,
           be2, Wl3, bl3, Wr3, g3, be3, fc1_w, fc1_b, fc2_w, fc2_b):
  src = jnp.concatenate(
      [edge_index[0], jnp.zeros((E_PAD - E_EDGES,), jnp.int32)])
  dst = jnp.concatenate(
      [edge_index[1], jnp.full((E_PAD - E_EDGES,), N_PAD - 1, jnp.int32)])
  pad_n = N_PAD - x.shape[0]
  x_p = jnp.pad(x, ((0, pad_n), (0, 0)))
  batch_p = jnp.pad(batch, (0, pad_n), constant_values=G_POOL)
  zeros128 = jnp.zeros((N_PAD, 128), jnp.float32)

  cnt = _count(dst).reshape(NW, N_PAD)
  part1 = _agg128(x_p, src, dst, zeros128).reshape(NC, N_PAD, 128)
  h1 = _dense128(part1, cnt, x_p, Wl1, bl1, Wr1, g1, be1)
  part2 = _agg128(h1, src, dst, zeros128).reshape(NC, N_PAD, 128)
  h2 = _dense128(part2, cnt, h1, Wl2, bl2, Wr2, g2, be2)
  part3 = _agg128(h2, src, dst, zeros128).reshape(NC, N_PAD, 128)
  h3 = _dense64(part3, cnt, h2, Wl3, bl3, Wr3, g3, be3)
  return _pool(h3, batch_p, fc1_w, fc1_b, fc2_w, fc2_b)
